# baseline (speedup 1.0000x reference)
import functools

import jax
import jax.numpy as jnp
from jax import lax
from jax.experimental import pallas as pl
from jax.experimental.pallas import tpu as pltpu
from jax.experimental.pallas import tpu_sc as plsc

N_ROWS = 4096
N_COLS = 2048

_info = plsc.get_sparse_core_info()
_NC, _NS, _L = _info.num_cores, _info.num_subcores, _info.num_lanes
_NW = _NC * _NS
_CW = N_COLS // _NW
_G = _CW // _L
_R = 512
_NCHUNK = N_ROWS // _R

_mesh = plsc.VectorSubcoreMesh(core_axis_name="c", subcore_axis_name="s")


@functools.partial(
    pl.kernel,
    out_type=jax.ShapeDtypeStruct((N_ROWS, N_COLS), jnp.float32),
    mesh=_mesh,
    scratch_types=[pltpu.VMEM((_R, _CW), jnp.float32)],
    compiler_params=pltpu.CompilerParams(use_tc_tiling_on_sc=False),
)
def _cumsum_sc(x_hbm, out_hbm, buf):
    wid = lax.axis_index("s") * _NC + lax.axis_index("c")
    col0 = pl.multiple_of(wid * _CW, _CW)

    def chunk_body(ci, carries):
        r0 = pl.multiple_of(ci * _R, _R)
        pltpu.sync_copy(x_hbm.at[pl.ds(r0, _R), pl.ds(col0, _CW)], buf)

        def row_body(r, cs):
            new = []
            for g in range(_G):
                v = cs[g] + buf[r, pl.ds(g * _L, _L)]
                buf[r, pl.ds(g * _L, _L)] = v
                new.append(v)
            return tuple(new)

        carries = lax.fori_loop(0, _R, row_body, carries)
        pltpu.sync_copy(buf, out_hbm.at[pl.ds(r0, _R), pl.ds(col0, _CW)])
        return carries

    carry0 = tuple(jnp.zeros((_L,), jnp.float32) for _ in range(_G))
    lax.fori_loop(0, _NCHUNK, chunk_body, carry0)


def kernel(x):
    return _cumsum_sc(x)
